# speedup vs baseline: 5.3464x; 5.3464x over previous
import jax
import jax.numpy as jnp
from jax import lax
from jax.experimental import pallas as pl
from jax.experimental.pallas import tpu as pltpu
from jax.experimental.pallas import tpu_sc as plsc

TOTAL_NODES = 131072
H = 128
B = 1024

NC = 2
NS = 16
NW = NC * NS
ROWS_PER_TILE = TOTAL_NODES // NW
CHUNK = 128
NCHUNK = ROWS_PER_TILE // CHUNK
BALL_PER_TILE = B // NW
ACC_PER_TILE = B // NS


def _sc_body(nodes, bidx, bptr, zeros, part_out, cnt_out, ball_out,
             idx_v, rows_v, ones_v, zrow_v, zcnt_v, bptr_v, ball_v,
             acc_v, cntr_v, acc_sh, cnt_sh, sem):
  c = lax.axis_index("c")
  s = lax.axis_index("s")
  wid = c * NS + s

  pltpu.sync_copy(zeros.at[pl.ds(s * ACC_PER_TILE, ACC_PER_TILE)], zrow_v)
  pltpu.sync_copy(zrow_v, acc_sh.at[pl.ds(s * ACC_PER_TILE, ACC_PER_TILE)])
  pltpu.sync_copy(zeros.at[s, pl.ds(0, ACC_PER_TILE)], zcnt_v)
  pltpu.sync_copy(zcnt_v, cnt_sh.at[pl.ds(s * ACC_PER_TILE, ACC_PER_TILE)])

  pltpu.sync_copy(bptr.at[pl.ds(wid * BALL_PER_TILE, BALL_PER_TILE)], bptr_v)
  pltpu.async_copy(nodes.at[bptr_v], ball_v, sem).wait()
  pltpu.sync_copy(ball_v, ball_out.at[pl.ds(wid * BALL_PER_TILE, BALL_PER_TILE)])

  for i in range(CHUNK // 16):
    ones_v[pl.ds(i * 16, 16)] = jnp.ones((16,), jnp.float32)

  pltpu.sync_copy(bidx.at[pl.ds(wid * NCHUNK, NCHUNK)], idx_v)

  plsc.subcore_barrier()

  row0 = wid * ROWS_PER_TILE

  @pl.loop(0, NCHUNK)
  def _(j):
    pltpu.sync_copy(nodes.at[pl.ds(row0 + j * CHUNK, CHUNK)], rows_v)
    pltpu.sync_copy(rows_v, acc_sh.at[idx_v.at[j]], add=True)
    pltpu.sync_copy(ones_v, cnt_sh.at[idx_v.at[j]], add=True)

  plsc.subcore_barrier()

  pltpu.sync_copy(acc_sh.at[pl.ds(s * ACC_PER_TILE, ACC_PER_TILE)], acc_v)
  pltpu.sync_copy(acc_v, part_out.at[c, pl.ds(s * ACC_PER_TILE, ACC_PER_TILE)])
  pltpu.sync_copy(cnt_sh.at[pl.ds(s * ACC_PER_TILE, ACC_PER_TILE)], cntr_v)
  pltpu.sync_copy(cntr_v, cnt_out.at[c, pl.ds(s * ACC_PER_TILE, ACC_PER_TILE)])


@jax.jit
def _sc_aggregate(nodes, bidx2d, bptr, zeros):
  mesh = plsc.VectorSubcoreMesh(core_axis_name="c", subcore_axis_name="s")
  return pl.kernel(
      _sc_body,
      out_type=(
          jax.ShapeDtypeStruct((NC, B, H), jnp.float32),
          jax.ShapeDtypeStruct((NC, B), jnp.float32),
          jax.ShapeDtypeStruct((B, H), jnp.float32),
      ),
      mesh=mesh,
      scratch_types=[
          pltpu.VMEM((NCHUNK, CHUNK), jnp.int32),
          pltpu.VMEM((CHUNK, H), jnp.float32),
          pltpu.VMEM((CHUNK,), jnp.float32),
          pltpu.VMEM((ACC_PER_TILE, H), jnp.float32),
          pltpu.VMEM((ACC_PER_TILE,), jnp.float32),
          pltpu.VMEM((BALL_PER_TILE,), jnp.int32),
          pltpu.VMEM((BALL_PER_TILE, H), jnp.float32),
          pltpu.VMEM((ACC_PER_TILE, H), jnp.float32),
          pltpu.VMEM((ACC_PER_TILE,), jnp.float32),
          pltpu.VMEM_SHARED((B, H), jnp.float32),
          pltpu.VMEM_SHARED((B,), jnp.float32),
          pltpu.SemaphoreType.DMA,
      ],
  )(nodes, bidx2d, bptr, zeros)


def _tc_head(part_ref, cnt_ref, ball_ref, g_ref, bb_ref, w1_ref, b1_ref,
             w2_ref, b2_ref, out_ref):
  part = part_ref[...]
  seg = part[0] + part[1]
  cnt = jnp.sum(cnt_ref[...], axis=1, keepdims=True)
  ge = seg / jnp.maximum(cnt, 1.0)
  f = jnp.concatenate([ball_ref[...], ge], axis=1)
  mu = jnp.mean(f, axis=1, keepdims=True)
  d = f - mu
  var = jnp.mean(d * d, axis=1, keepdims=True)
  h = d * lax.rsqrt(var + 1e-5) * g_ref[...] + bb_ref[...]
  h = jnp.maximum(
      jnp.dot(h, w1_ref[...], preferred_element_type=jnp.float32)
      + b1_ref[...], 0.0)
  out_ref[...] = (
      jnp.dot(h, w2_ref[...], preferred_element_type=jnp.float32)
      + b2_ref[...])


@jax.jit
def _tc_finish(part, cnt2t, ball, ln_g, ln_b, W1, b1, W2, b2):
  return pl.pallas_call(
      _tc_head,
      out_shape=jax.ShapeDtypeStruct((B, H), jnp.float32),
  )(part, cnt2t, ball, ln_g, ln_b, W1, b1, W2, b2)


def kernel(node_emb, batch_ptr, batch_index, ln_g, ln_b, W1, b1, W2, b2):
  bidx2d = batch_index.astype(jnp.int32).reshape(NW * NCHUNK, CHUNK)
  bptr = batch_ptr[:-1].astype(jnp.int32)
  zeros = jnp.zeros((B, H), jnp.float32)
  part, cnt2, ball = _sc_aggregate(node_emb, bidx2d, bptr, zeros)
  return _tc_finish(part, cnt2.T, ball,
                    ln_g.reshape(1, 2 * H), ln_b.reshape(1, 2 * H),
                    W1, b1.reshape(1, H), W2, b2.reshape(1, H))


# speedup vs baseline: 6.5994x; 1.2344x over previous
import jax
import jax.numpy as jnp
from jax import lax
from jax.experimental import pallas as pl
from jax.experimental.pallas import tpu as pltpu
from jax.experimental.pallas import tpu_sc as plsc

TOTAL_NODES = 131072
H = 128
B = 1024

NC = 2
NS = 16
NW = NC * NS
ROWS_PER_TILE = TOTAL_NODES // NW
CHUNK = 128
NCHUNK = ROWS_PER_TILE // CHUNK
BALL_PER_TILE = B // NW
ACC_PER_TILE = B // NS


def _sc_body(nodes, bidx, bptr, zeros, part_out, cnt_out, ball_out,
             idx_v, rows_v, ones_v, zrow_v, zcnt_v, bptr_v, ball_v,
             acc_v, cntr_v, acc_sh, cnt_sh, sem, sem0, sem1):
  c = lax.axis_index("c")
  s = lax.axis_index("s")
  wid = c * NS + s

  pltpu.sync_copy(zeros.at[pl.ds(s * ACC_PER_TILE, ACC_PER_TILE)], zrow_v)
  pltpu.sync_copy(zrow_v, acc_sh.at[pl.ds(s * ACC_PER_TILE, ACC_PER_TILE)])
  pltpu.sync_copy(zeros.at[s, pl.ds(0, ACC_PER_TILE)], zcnt_v)
  pltpu.sync_copy(zcnt_v, cnt_sh.at[pl.ds(s * ACC_PER_TILE, ACC_PER_TILE)])

  pltpu.sync_copy(bptr.at[pl.ds(wid * BALL_PER_TILE, BALL_PER_TILE)], bptr_v)
  pltpu.async_copy(nodes.at[bptr_v], ball_v, sem).wait()
  pltpu.sync_copy(ball_v, ball_out.at[pl.ds(wid * BALL_PER_TILE, BALL_PER_TILE)])

  for i in range(CHUNK // 16):
    ones_v[pl.ds(i * 16, 16)] = jnp.ones((16,), jnp.float32)

  pltpu.sync_copy(bidx.at[pl.ds(wid * NCHUNK, NCHUNK)], idx_v)

  plsc.subcore_barrier()

  row0 = wid * ROWS_PER_TILE

  pltpu.async_copy(nodes.at[pl.ds(row0, CHUNK)], rows_v.at[0], sem0)

  @pl.loop(0, NCHUNK, step=2)
  def _(j):
    pltpu.async_copy(
        nodes.at[pl.ds(row0 + (j + 1) * CHUNK, CHUNK)], rows_v.at[1], sem1)
    pltpu.make_async_copy(
        nodes.at[pl.ds(row0, CHUNK)], rows_v.at[0], sem0).wait()
    pltpu.sync_copy(rows_v.at[0], acc_sh.at[idx_v.at[j]], add=True)
    pltpu.sync_copy(ones_v, cnt_sh.at[idx_v.at[j]], add=True)

    @pl.when(j + 2 < NCHUNK)
    def _():
      pltpu.async_copy(
          nodes.at[pl.ds(row0 + (j + 2) * CHUNK, CHUNK)], rows_v.at[0], sem0)

    pltpu.make_async_copy(
        nodes.at[pl.ds(row0, CHUNK)], rows_v.at[1], sem1).wait()
    pltpu.sync_copy(rows_v.at[1], acc_sh.at[idx_v.at[j + 1]], add=True)
    pltpu.sync_copy(ones_v, cnt_sh.at[idx_v.at[j + 1]], add=True)

  plsc.subcore_barrier()

  pltpu.sync_copy(acc_sh.at[pl.ds(s * ACC_PER_TILE, ACC_PER_TILE)], acc_v)
  pltpu.sync_copy(acc_v, part_out.at[c, pl.ds(s * ACC_PER_TILE, ACC_PER_TILE)])
  pltpu.sync_copy(cnt_sh.at[pl.ds(s * ACC_PER_TILE, ACC_PER_TILE)], cntr_v)
  pltpu.sync_copy(cntr_v, cnt_out.at[c, pl.ds(s * ACC_PER_TILE, ACC_PER_TILE)])


@jax.jit
def _sc_aggregate(nodes, bidx2d, bptr, zeros):
  mesh = plsc.VectorSubcoreMesh(core_axis_name="c", subcore_axis_name="s")
  return pl.kernel(
      _sc_body,
      out_type=(
          jax.ShapeDtypeStruct((NC, B, H), jnp.float32),
          jax.ShapeDtypeStruct((NC, B), jnp.float32),
          jax.ShapeDtypeStruct((B, H), jnp.float32),
      ),
      mesh=mesh,
      scratch_types=[
          pltpu.VMEM((NCHUNK, CHUNK), jnp.int32),
          pltpu.VMEM((2, CHUNK, H), jnp.float32),
          pltpu.VMEM((CHUNK,), jnp.float32),
          pltpu.VMEM((ACC_PER_TILE, H), jnp.float32),
          pltpu.VMEM((ACC_PER_TILE,), jnp.float32),
          pltpu.VMEM((BALL_PER_TILE,), jnp.int32),
          pltpu.VMEM((BALL_PER_TILE, H), jnp.float32),
          pltpu.VMEM((ACC_PER_TILE, H), jnp.float32),
          pltpu.VMEM((ACC_PER_TILE,), jnp.float32),
          pltpu.VMEM_SHARED((B, H), jnp.float32),
          pltpu.VMEM_SHARED((B,), jnp.float32),
          pltpu.SemaphoreType.DMA,
          pltpu.SemaphoreType.DMA,
          pltpu.SemaphoreType.DMA,
      ],
  )(nodes, bidx2d, bptr, zeros)


def _tc_head(part_ref, cnt_ref, ball_ref, g_ref, bb_ref, w1_ref, b1_ref,
             w2_ref, b2_ref, out_ref):
  part = part_ref[...]
  seg = part[0] + part[1]
  cnt = jnp.sum(cnt_ref[...], axis=1, keepdims=True)
  ge = seg / jnp.maximum(cnt, 1.0)
  f = jnp.concatenate([ball_ref[...], ge], axis=1)
  mu = jnp.mean(f, axis=1, keepdims=True)
  d = f - mu
  var = jnp.mean(d * d, axis=1, keepdims=True)
  h = d * lax.rsqrt(var + 1e-5) * g_ref[...] + bb_ref[...]
  h = jnp.maximum(
      jnp.dot(h, w1_ref[...], preferred_element_type=jnp.float32)
      + b1_ref[...], 0.0)
  out_ref[...] = (
      jnp.dot(h, w2_ref[...], preferred_element_type=jnp.float32)
      + b2_ref[...])


@jax.jit
def _tc_finish(part, cnt2t, ball, ln_g, ln_b, W1, b1, W2, b2):
  return pl.pallas_call(
      _tc_head,
      out_shape=jax.ShapeDtypeStruct((B, H), jnp.float32),
  )(part, cnt2t, ball, ln_g, ln_b, W1, b1, W2, b2)


def kernel(node_emb, batch_ptr, batch_index, ln_g, ln_b, W1, b1, W2, b2):
  bidx2d = batch_index.astype(jnp.int32).reshape(NW * NCHUNK, CHUNK)
  bptr = batch_ptr[:-1].astype(jnp.int32)
  zeros = jnp.zeros((B, H), jnp.float32)
  part, cnt2, ball = _sc_aggregate(node_emb, bidx2d, bptr, zeros)
  return _tc_finish(part, cnt2.T, ball,
                    ln_g.reshape(1, 2 * H), ln_b.reshape(1, 2 * H),
                    W1, b1.reshape(1, H), W2, b2.reshape(1, H))
